# baseline (speedup 1.0000x reference)
import functools

import jax
import jax.numpy as jnp
from jax import lax
from jax.experimental import pallas as pl
from jax.experimental.pallas import tpu as pltpu
from jax.experimental.pallas import tpu_sc as plsc

_LANES = 16
_ROWS = 16


def kernel(inputs, embeddings):
    B, S, D = inputs.shape
    SD = S * D
    pos = embeddings[:S].reshape(SD)
    x = inputs.reshape(B * SD)

    info = plsc.get_sparse_core_info()
    NC, NS = info.num_cores, info.num_subcores
    NW = NC * NS

    CHUNK = _ROWS * D
    words_per_worker = SD // NW
    n_chunks = words_per_worker // CHUNK
    assert words_per_worker % CHUNK == 0 and n_chunks % 2 == 0
    n_pairs = n_chunks // 2
    n_vecs = CHUNK // _LANES

    mesh = plsc.VectorSubcoreMesh(core_axis_name="c", subcore_axis_name="s")

    @functools.partial(
        pl.kernel,
        mesh=mesh,
        out_type=jax.ShapeDtypeStruct((B * SD,), jnp.float32),
        scratch_types=[
            pltpu.VMEM((2, CHUNK), jnp.float32),
            pltpu.VMEM((2, B, CHUNK), jnp.float32),
            pltpu.SemaphoreType.DMA,
            pltpu.SemaphoreType.DMA,
            pltpu.SemaphoreType.DMA,
            pltpu.SemaphoreType.DMA,
        ],
    )
    def k(x_hbm, e_hbm, o_hbm, emb_v, in_v, sem_in0, sem_in1, sem_out0, sem_out1):
        sem_in = (sem_in0, sem_in1)
        sem_out = (sem_out0, sem_out1)
        wid = lax.axis_index("s") * NC + lax.axis_index("c")
        base = wid * words_per_worker

        def in_copies(c, slot):
            off = base + c * CHUNK
            cps = [pltpu.make_async_copy(
                e_hbm.at[pl.ds(off, CHUNK)], emb_v.at[slot], sem_in[slot])]
            for b in range(B):
                cps.append(pltpu.make_async_copy(
                    x_hbm.at[pl.ds(b * SD + off, CHUNK)],
                    in_v.at[slot, b], sem_in[slot]))
            return cps

        def out_copy(c, slot, b):
            off = base + c * CHUNK
            return pltpu.make_async_copy(
                in_v.at[slot, b], o_hbm.at[pl.ds(b * SD + off, CHUNK)],
                sem_out[slot])

        for cp in in_copies(0, 0):
            cp.start()

        def pair_body(it, carry):
            i2 = it * 2
            for j in (0, 1):
                c = i2 + j

                if j == 0:
                    @pl.when(c >= 1)
                    def _():
                        for b in range(B):
                            out_copy(c - 1, 1 - j, b).wait()
                else:
                    for b in range(B):
                        out_copy(c - 1, 1 - j, b).wait()

                if j == 0:
                    for cp in in_copies(c + 1, 1 - j):
                        cp.start()
                else:
                    @pl.when(it < n_pairs - 1)
                    def _():
                        for cp in in_copies(c + 1, 1 - j):
                            cp.start()

                for cp in in_copies(c, j):
                    cp.wait()

                for b in range(B):
                    def add_body(kk, carry3, _b=b, _j=j):
                        sl = pl.ds(kk * _LANES, _LANES)
                        plsc.addupdate(in_v.at[_j, _b, sl], emb_v[_j, sl])
                        return carry3

                    lax.fori_loop(0, n_vecs, add_body, 0, unroll=8)
                    out_copy(c, j, b).start()
            return carry

        lax.fori_loop(0, n_pairs, pair_body, 0)

        for b in range(B):
            out_copy(n_chunks - 1, 1, b).wait()

    out = k(x, pos)
    return out.reshape(B, S, D)


# speedup vs baseline: 1.3255x; 1.3255x over previous
import functools

import jax
import jax.numpy as jnp
from jax import lax
from jax.experimental import pallas as pl
from jax.experimental.pallas import tpu as pltpu
from jax.experimental.pallas import tpu_sc as plsc

_LANES = 16
_ROWS = 16


def kernel(inputs, embeddings):
    B, S, D = inputs.shape
    SD = S * D
    pos = embeddings[:S].reshape(SD)
    x = inputs.reshape(B * SD)

    info = plsc.get_sparse_core_info()
    NC, NS = info.num_cores, info.num_subcores
    NW = NC * NS

    CHUNK = _ROWS * D
    words_per_worker = SD // NW
    n_chunks = words_per_worker // CHUNK
    assert words_per_worker % CHUNK == 0 and n_chunks % 2 == 0
    n_pairs = n_chunks // 2
    n_vecs = CHUNK // _LANES

    mesh = plsc.VectorSubcoreMesh(core_axis_name="c", subcore_axis_name="s")

    @functools.partial(
        pl.kernel,
        mesh=mesh,
        out_type=jax.ShapeDtypeStruct((B * SD,), jnp.float32),
        scratch_types=[
            pltpu.VMEM((2, CHUNK), jnp.float32),
            pltpu.VMEM((2, B, CHUNK), jnp.float32),
            pltpu.SemaphoreType.DMA,
            pltpu.SemaphoreType.DMA,
            pltpu.SemaphoreType.DMA,
            pltpu.SemaphoreType.DMA,
        ],
    )
    def k(x_hbm, e_hbm, o_hbm, emb_v, in_v, sem_in0, sem_in1, sem_out0, sem_out1):
        sem_in = (sem_in0, sem_in1)
        sem_out = (sem_out0, sem_out1)
        wid = lax.axis_index("s") * NC + lax.axis_index("c")
        base = wid * words_per_worker

        def in_copies(c, slot):
            off = base + c * CHUNK
            cps = [pltpu.make_async_copy(
                e_hbm.at[pl.ds(off, CHUNK)], emb_v.at[slot], sem_in[slot])]
            for b in range(B):
                cps.append(pltpu.make_async_copy(
                    x_hbm.at[pl.ds(b * SD + off, CHUNK)],
                    in_v.at[slot, b], sem_in[slot]))
            return cps

        def out_copy(c, slot, b):
            off = base + c * CHUNK
            return pltpu.make_async_copy(
                in_v.at[slot, b], o_hbm.at[pl.ds(b * SD + off, CHUNK)],
                sem_out[slot])

        for cp in in_copies(0, 0):
            cp.start()

        def pair_body(it, carry):
            i2 = it * 2
            for j in (0, 1):
                c = i2 + j

                if j == 0:
                    @pl.when(c >= 1)
                    def _():
                        for b in range(B):
                            out_copy(c - 1, 1 - j, b).wait()
                else:
                    for b in range(B):
                        out_copy(c - 1, 1 - j, b).wait()

                if j == 0:
                    for cp in in_copies(c + 1, 1 - j):
                        cp.start()
                else:
                    @pl.when(it < n_pairs - 1)
                    def _():
                        for cp in in_copies(c + 1, 1 - j):
                            cp.start()

                for cp in in_copies(c, j):
                    cp.wait()

                for b in range(B):
                    @plsc.parallel_loop(0, n_vecs, unroll=8)
                    def add_body(kk, _b=b, _j=j):
                        sl = pl.ds(kk * _LANES, _LANES)
                        plsc.addupdate(in_v.at[_j, _b, sl], emb_v[_j, sl])

                    out_copy(c, j, b).start()
            return carry

        lax.fori_loop(0, n_pairs, pair_body, 0)

        for b in range(B):
            out_copy(n_chunks - 1, 1, b).wait()

    out = k(x, pos)
    return out.reshape(B, S, D)


# speedup vs baseline: 3.5408x; 2.6712x over previous
import functools

import jax
import jax.numpy as jnp
from jax import lax
from jax.experimental import pallas as pl
from jax.experimental.pallas import tpu as pltpu
from jax.experimental.pallas import tpu_sc as plsc

_LANES = 16
_ROWS = 16


def kernel(inputs, embeddings):
    B, S, D = inputs.shape
    assert D % _LANES == 0

    info = plsc.get_sparse_core_info()
    NC, NS = info.num_cores, info.num_subcores
    NW = NC * NS

    rows_per_worker = S // NW
    n_chunks = rows_per_worker // _ROWS
    assert S % NW == 0 and rows_per_worker % _ROWS == 0 and n_chunks % 2 == 0
    n_pairs = n_chunks // 2
    n_col_vecs = D // _LANES

    mesh = plsc.VectorSubcoreMesh(core_axis_name="c", subcore_axis_name="s")

    @functools.partial(
        pl.kernel,
        mesh=mesh,
        out_type=jax.ShapeDtypeStruct((B, S, D), jnp.float32),
        scratch_types=[
            pltpu.VMEM((2, _ROWS, D), jnp.float32),
            pltpu.VMEM((2, B, _ROWS, D), jnp.float32),
            pltpu.SemaphoreType.DMA,
            pltpu.SemaphoreType.DMA,
            pltpu.SemaphoreType.DMA,
            pltpu.SemaphoreType.DMA,
        ],
    )
    def k(x_hbm, e_hbm, o_hbm, emb_v, in_v, sem_in0, sem_in1, sem_out0, sem_out1):
        sem_in = (sem_in0, sem_in1)
        sem_out = (sem_out0, sem_out1)
        wid = lax.axis_index("s") * NC + lax.axis_index("c")
        base_row = wid * rows_per_worker

        def in_copies(c, slot):
            r0 = base_row + c * _ROWS
            cps = [pltpu.make_async_copy(
                e_hbm.at[pl.ds(r0, _ROWS)], emb_v.at[slot], sem_in[slot])]
            for b in range(B):
                cps.append(pltpu.make_async_copy(
                    x_hbm.at[b, pl.ds(r0, _ROWS)],
                    in_v.at[slot, b], sem_in[slot]))
            return cps

        def out_copy(c, slot, b):
            r0 = base_row + c * _ROWS
            return pltpu.make_async_copy(
                in_v.at[slot, b], o_hbm.at[b, pl.ds(r0, _ROWS)],
                sem_out[slot])

        for cp in in_copies(0, 0):
            cp.start()

        def pair_body(it, carry):
            i2 = it * 2
            for j in (0, 1):
                c = i2 + j

                if j == 0:
                    @pl.when(c >= 1)
                    def _():
                        for b in range(B):
                            out_copy(c - 1, 1 - j, b).wait()
                else:
                    for b in range(B):
                        out_copy(c - 1, 1 - j, b).wait()

                if j == 0:
                    for cp in in_copies(c + 1, 1 - j):
                        cp.start()
                else:
                    @pl.when(it < n_pairs - 1)
                    def _():
                        for cp in in_copies(c + 1, 1 - j):
                            cp.start()

                for cp in in_copies(c, j):
                    cp.wait()

                for b in range(B):
                    @plsc.parallel_loop(0, _ROWS, unroll=2)
                    def add_body(row, _b=b, _j=j):
                        for u in range(n_col_vecs):
                            sl = pl.ds(u * _LANES, _LANES)
                            plsc.addupdate(
                                in_v.at[_j, _b, row, sl], emb_v[_j, row, sl])

                    out_copy(c, j, b).start()
            return carry

        lax.fori_loop(0, n_pairs, pair_body, 0)

        for b in range(B):
            out_copy(n_chunks - 1, 1, b).wait()

    return k(inputs, embeddings)


# speedup vs baseline: 4.4219x; 1.2488x over previous
import functools

import jax
import jax.numpy as jnp
from jax import lax
from jax.experimental import pallas as pl
from jax.experimental.pallas import tpu as pltpu
from jax.experimental.pallas import tpu_sc as plsc

_LANES = 16
_ROWS = 8
_NSLOTS = 4
_AHEAD = 2


def kernel(inputs, embeddings):
    B, S, D = inputs.shape
    assert D % _LANES == 0

    info = plsc.get_sparse_core_info()
    NC, NS = info.num_cores, info.num_subcores
    NW = NC * NS

    rows_per_worker = S // NW
    n_chunks = rows_per_worker // _ROWS
    assert S % NW == 0 and rows_per_worker % _ROWS == 0
    assert n_chunks % _NSLOTS == 0
    n_groups = n_chunks // _NSLOTS
    n_col_vecs = D // _LANES

    mesh = plsc.VectorSubcoreMesh(core_axis_name="c", subcore_axis_name="s")

    @functools.partial(
        pl.kernel,
        mesh=mesh,
        out_type=jax.ShapeDtypeStruct((B, S, D), jnp.float32),
        scratch_types=[
            pltpu.VMEM((_NSLOTS, _ROWS, D), jnp.float32),
            pltpu.VMEM((_NSLOTS, B, _ROWS, D), jnp.float32),
            pltpu.SemaphoreType.DMA,
            pltpu.SemaphoreType.DMA,
            pltpu.SemaphoreType.DMA,
            pltpu.SemaphoreType.DMA,
            pltpu.SemaphoreType.DMA,
            pltpu.SemaphoreType.DMA,
            pltpu.SemaphoreType.DMA,
            pltpu.SemaphoreType.DMA,
        ],
    )
    def k(x_hbm, e_hbm, o_hbm, emb_v, in_v,
          si0, si1, si2, si3, so0, so1, so2, so3):
        sem_in = (si0, si1, si2, si3)
        sem_out = (so0, so1, so2, so3)
        wid = lax.axis_index("s") * NC + lax.axis_index("c")
        base_row = wid * rows_per_worker

        def in_copies(c, slot):
            r0 = base_row + c * _ROWS
            cps = [pltpu.make_async_copy(
                e_hbm.at[pl.ds(r0, _ROWS)], emb_v.at[slot], sem_in[slot])]
            for b in range(B):
                cps.append(pltpu.make_async_copy(
                    x_hbm.at[b, pl.ds(r0, _ROWS)],
                    in_v.at[slot, b], sem_in[slot]))
            return cps

        def out_copy(c, slot, b):
            r0 = base_row + c * _ROWS
            return pltpu.make_async_copy(
                in_v.at[slot, b], o_hbm.at[b, pl.ds(r0, _ROWS)],
                sem_out[slot])

        for c0 in range(_AHEAD):
            for cp in in_copies(c0, c0):
                cp.start()

        def group_body(it, carry):
            i0 = it * _NSLOTS
            for j in range(_NSLOTS):
                c = i0 + j
                nslot = (j + _AHEAD) % _NSLOTS

                if j + _AHEAD >= _NSLOTS:
                    for b in range(B):
                        out_copy(c + _AHEAD - _NSLOTS, nslot, b).wait()

                    @pl.when(it < n_groups - 1)
                    def _():
                        for cp in in_copies(c + _AHEAD, nslot):
                            cp.start()
                else:
                    @pl.when(c + _AHEAD - _NSLOTS >= 0)
                    def _():
                        for b in range(B):
                            out_copy(c + _AHEAD - _NSLOTS, nslot, b).wait()
                    for cp in in_copies(c + _AHEAD, nslot):
                        cp.start()

                for cp in in_copies(c, j):
                    cp.wait()

                for b in range(B):
                    @plsc.parallel_loop(0, _ROWS, unroll=1)
                    def add_body(row, _b=b, _j=j):
                        for u in range(n_col_vecs):
                            sl = pl.ds(u * _LANES, _LANES)
                            plsc.addupdate(
                                in_v.at[_j, _b, row, sl], emb_v[_j, row, sl])

                    out_copy(c, j, b).start()
            return carry

        lax.fori_loop(0, n_groups, group_body, 0)

        for d in range(_NSLOTS - _AHEAD):
            c = n_chunks - (_NSLOTS - _AHEAD) + d
            for b in range(B):
                out_copy(c, c % _NSLOTS, b).wait()

    return k(inputs, embeddings)
